# baseline (speedup 1.0000x reference)
import jax
import jax.numpy as jnp
from jax import lax
from jax.experimental import pallas as pl
from jax.experimental.pallas import tpu as pltpu
from jax.experimental.pallas import tpu_sc as plsc

_E = 1600000
_NFEAT = 4
_ROWS = 16
_DIM = 32
_NC, _NS, _L = 2, 16, 16
_NW = _NC * _NS
_EPW = _E // _NW
_UN = 5
_CH = 10000
_NCH = _EPW // _CH
_NBUF = 6
_DEPTH = 4


def _sc_body(f0, f1, f2, f3, tabs, out, b0, b1, b2, b3, b4, b5, tab_v,
             acc_v, s0, s1, s2, s3, s4, s5, semt):
    cid = lax.axis_index("c")
    sid = lax.axis_index("s")
    wid = sid * _NC + cid
    base = wid * _EPW

    feats = [f0, f1, f2, f3]
    sems = [s0, s1, s2, s3, s4, s5]
    bufs = [b0, b1, b2, b3, b4, b5]

    def chunk_src(k):
        f, c = divmod(k, _NCH)
        return feats[f].at[pl.ds(base + c * _CH, _CH)]

    total = _NFEAT * _NCH
    copies = [None] * total
    for k in range(_DEPTH):
        copies[k] = pltpu.async_copy(chunk_src(k), bufs[k % _NBUF],
                                     sems[k % _NBUF])

    pltpu.async_copy(tabs, tab_v, semt).wait()
    lanes = lax.iota(jnp.int32, _L).astype(jnp.float32)
    s_list = []
    for f in range(_NFEAT):
        s = tab_v[f * _DIM, :]
        for r in range(1, _DIM):
            s = s + tab_v[f * _DIM + r, :]
        s_list.append(s * lanes)

    accs = tuple(jnp.zeros((_L,), jnp.float32) for _ in range(_UN))
    for k in range(total):
        if k + _DEPTH < total:
            kk = k + _DEPTH
            copies[kk] = pltpu.async_copy(chunk_src(kk), bufs[kk % _NBUF],
                                          sems[kk % _NBUF])
        copies[k].wait()
        s = s_list[k // _NCH]
        buf = bufs[k % _NBUF]

        @plsc.parallel_loop(0, _CH, step=_UN * _L, unroll=5, carry=accs)
        def body(i, accs, buf=buf, s=s):
            new = []
            for j in range(_UN):
                v = buf[pl.ds(i + j * _L, _L)]
                idx = lax.bitcast_convert_type(v + jnp.float32(8388608.0),
                                               jnp.int32)
                new.append(accs[j] + s.at[idx].get(mode="promise_in_bounds"))
            return tuple(new)

        accs = body

    acc = accs[0]
    for j in range(1, _UN):
        acc = acc + accs[j]

    acc_v[:] = acc
    pltpu.sync_copy(acc_v, out.at[wid])


def kernel(bond_type, stereo, is_conjugated, is_in_ring,
           table_bond_type, table_stereo, table_is_conjugated,
           table_is_in_ring):
    mesh = plsc.VectorSubcoreMesh(core_axis_name="c", subcore_axis_name="s")
    run = pl.kernel(
        _sc_body,
        out_type=jax.ShapeDtypeStruct((_NW, _L), jnp.float32),
        mesh=mesh,
        scratch_types=[
            pltpu.VMEM((_CH,), jnp.float32),
            pltpu.VMEM((_CH,), jnp.float32),
            pltpu.VMEM((_CH,), jnp.float32),
            pltpu.VMEM((_CH,), jnp.float32),
            pltpu.VMEM((_CH,), jnp.float32),
            pltpu.VMEM((_CH,), jnp.float32),
            pltpu.VMEM((_NFEAT * _DIM, _L), jnp.float32),
            pltpu.VMEM((_L,), jnp.float32),
            pltpu.SemaphoreType.DMA,
            pltpu.SemaphoreType.DMA,
            pltpu.SemaphoreType.DMA,
            pltpu.SemaphoreType.DMA,
            pltpu.SemaphoreType.DMA,
            pltpu.SemaphoreType.DMA,
            pltpu.SemaphoreType.DMA,
        ],
    )
    tabs = jnp.concatenate(
        [table_bond_type.T, table_stereo.T, table_is_conjugated.T,
         table_is_in_ring.T], axis=0)
    partials = run(bond_type, stereo, is_conjugated, is_in_ring, tabs)
    return jnp.sum(partials)


# speedup vs baseline: 1.0063x; 1.0063x over previous
import jax
import jax.numpy as jnp
from jax import lax
from jax.experimental import pallas as pl
from jax.experimental.pallas import tpu as pltpu
from jax.experimental.pallas import tpu_sc as plsc

_E = 1600000
_NFEAT = 4
_ROWS = 16
_DIM = 32
_NC, _NS, _L = 2, 16, 16
_NW = _NC * _NS
_EPW = _E // _NW
_UN = 5


def _sc_body(f0, f1, f2, f3, tabs, out, buf0, buf1, tab_v, acc_v,
             sem0, sem1, semt):
    cid = lax.axis_index("c")
    sid = lax.axis_index("s")
    wid = sid * _NC + cid
    base = wid * _EPW

    feats = [f0, f1, f2, f3]
    sems = [sem0, sem1]
    bufs = [buf0, buf1]
    copies = [None, None]
    copies[0] = pltpu.async_copy(feats[0].at[pl.ds(base, _EPW)], bufs[0],
                                 sems[0])

    pltpu.async_copy(tabs, tab_v, semt).wait()
    lanes = lax.iota(jnp.int32, _L).astype(jnp.float32)
    s_list = []
    for f in range(_NFEAT):
        s = tab_v[f * _DIM, :]
        for r in range(1, _DIM):
            s = s + tab_v[f * _DIM + r, :]
        s_list.append(s * lanes)

    accs = tuple(jnp.zeros((_L,), jnp.float32) for _ in range(_UN))
    for f in range(_NFEAT):
        b = f % 2
        if f + 1 < _NFEAT:
            nb = (f + 1) % 2
            copies[nb] = pltpu.async_copy(
                feats[f + 1].at[pl.ds(base, _EPW)], bufs[nb], sems[nb])
        copies[b].wait()
        s = s_list[f]
        buf = bufs[b]

        @plsc.parallel_loop(0, _EPW, step=_UN * _L, unroll=5, carry=accs)
        def body(i, accs, buf=buf, s=s):
            new = []
            for j in range(_UN):
                v = buf[pl.ds(i + j * _L, _L)]
                idx = lax.bitcast_convert_type(v + jnp.float32(8388608.0),
                                               jnp.int32)
                new.append(accs[j] + s.at[idx].get(mode="promise_in_bounds"))
            return tuple(new)

        accs = body

    acc = accs[0]
    for j in range(1, _UN):
        acc = acc + accs[j]

    acc_v[:] = acc
    pltpu.sync_copy(acc_v, out.at[wid])


def kernel(bond_type, stereo, is_conjugated, is_in_ring,
           table_bond_type, table_stereo, table_is_conjugated,
           table_is_in_ring):
    mesh = plsc.VectorSubcoreMesh(core_axis_name="c", subcore_axis_name="s")
    run = pl.kernel(
        _sc_body,
        out_type=jax.ShapeDtypeStruct((_NW, _L), jnp.float32),
        mesh=mesh,
        scratch_types=[
            pltpu.VMEM((_EPW,), jnp.float32),
            pltpu.VMEM((_EPW,), jnp.float32),
            pltpu.VMEM((_NFEAT * _DIM, _L), jnp.float32),
            pltpu.VMEM((_L,), jnp.float32),
            pltpu.SemaphoreType.DMA,
            pltpu.SemaphoreType.DMA,
            pltpu.SemaphoreType.DMA,
        ],
    )
    tabs = jnp.concatenate(
        [table_bond_type.T, table_stereo.T, table_is_conjugated.T,
         table_is_in_ring.T], axis=0)
    partials = run(bond_type, stereo, is_conjugated, is_in_ring, tabs)
    return jnp.sum(partials)


# speedup vs baseline: 1.0148x; 1.0085x over previous
import jax
import jax.numpy as jnp
from jax import lax
from jax.experimental import pallas as pl
from jax.experimental.pallas import tpu as pltpu
from jax.experimental.pallas import tpu_sc as plsc

_E = 1600000
_NFEAT = 4
_ROWS = 16
_DIM = 32
_NC, _NS, _L = 2, 16, 16
_NW = _NC * _NS
_EPW = _E // _NW
_UN = 5


def _sc_body(f0, f1, f2, f3, t0, t1, t2, t3, out, buf0, buf1,
             tb0, tb1, tb2, tb3, acc_v, sem0, sem1, semt):
    cid = lax.axis_index("c")
    sid = lax.axis_index("s")
    wid = sid * _NC + cid
    base = wid * _EPW

    feats = [f0, f1, f2, f3]
    sems = [sem0, sem1]
    bufs = [buf0, buf1]
    copies = [None, None]
    copies[0] = pltpu.async_copy(feats[0].at[pl.ds(base, _EPW)], bufs[0],
                                 sems[0])

    tabs_v = [tb0, tb1, tb2, tb3]
    tcopies = [pltpu.async_copy(t, tv, semt)
               for t, tv in zip([t0, t1, t2, t3], tabs_v)]
    for c in tcopies:
        c.wait()
    lane_i = lax.iota(jnp.int32, _L)
    lanes = lane_i.astype(jnp.float32)
    s_list = []
    for f in range(_NFEAT):
        tv = tabs_v[f]
        s = jnp.zeros((_L,), jnp.float32)
        for i in range(_ROWS):
            r = tv[i, pl.ds(0, _L)] + tv[i, pl.ds(_L, _L)]
            for sh in (8, 4, 2, 1):
                r = r + r.at[lane_i ^ sh].get(mode="promise_in_bounds")
            s = jnp.where(lane_i == i, r, s)
        s_list.append(s * lanes)

    accs = tuple(jnp.zeros((_L,), jnp.float32) for _ in range(_UN))
    for f in range(_NFEAT):
        b = f % 2
        if f + 1 < _NFEAT:
            nb = (f + 1) % 2
            copies[nb] = pltpu.async_copy(
                feats[f + 1].at[pl.ds(base, _EPW)], bufs[nb], sems[nb])
        copies[b].wait()
        s = s_list[f]
        buf = bufs[b]

        @plsc.parallel_loop(0, _EPW, step=_UN * _L, unroll=5, carry=accs)
        def body(i, accs, buf=buf, s=s):
            new = []
            for j in range(_UN):
                v = buf[pl.ds(i + j * _L, _L)]
                idx = lax.bitcast_convert_type(v + jnp.float32(8388608.0),
                                               jnp.int32)
                new.append(accs[j] + s.at[idx].get(mode="promise_in_bounds"))
            return tuple(new)

        accs = body

    acc = accs[0]
    for j in range(1, _UN):
        acc = acc + accs[j]

    acc_v[:] = acc
    pltpu.sync_copy(acc_v, out.at[wid])


def kernel(bond_type, stereo, is_conjugated, is_in_ring,
           table_bond_type, table_stereo, table_is_conjugated,
           table_is_in_ring):
    mesh = plsc.VectorSubcoreMesh(core_axis_name="c", subcore_axis_name="s")
    run = pl.kernel(
        _sc_body,
        out_type=jax.ShapeDtypeStruct((_NW, _L), jnp.float32),
        mesh=mesh,
        scratch_types=[
            pltpu.VMEM((_EPW,), jnp.float32),
            pltpu.VMEM((_EPW,), jnp.float32),
            pltpu.VMEM((_ROWS, _DIM), jnp.float32),
            pltpu.VMEM((_ROWS, _DIM), jnp.float32),
            pltpu.VMEM((_ROWS, _DIM), jnp.float32),
            pltpu.VMEM((_ROWS, _DIM), jnp.float32),
            pltpu.VMEM((_L,), jnp.float32),
            pltpu.SemaphoreType.DMA,
            pltpu.SemaphoreType.DMA,
            pltpu.SemaphoreType.DMA,
        ],
    )
    partials = run(bond_type, stereo, is_conjugated, is_in_ring,
                   table_bond_type, table_stereo, table_is_conjugated,
                   table_is_in_ring)
    return jnp.sum(partials)


# speedup vs baseline: 1.0181x; 1.0032x over previous
import jax
import jax.numpy as jnp
from jax import lax
from jax.experimental import pallas as pl
from jax.experimental.pallas import tpu as pltpu
from jax.experimental.pallas import tpu_sc as plsc

_E = 1600000
_NFEAT = 4
_ROWS = 16
_DIM = 32
_NC, _NS, _L = 2, 16, 16
_NW = _NC * _NS
_EPW = _E // _NW
_UN = 25


def _sc_body(f0, f1, f2, f3, t0, t1, t2, t3, out, buf0, buf1,
             tb0, tb1, tb2, tb3, acc_v, sem0, sem1, semt):
    cid = lax.axis_index("c")
    sid = lax.axis_index("s")
    wid = sid * _NC + cid
    base = wid * _EPW

    feats = [f0, f1, f2, f3]
    sems = [sem0, sem1]
    bufs = [buf0, buf1]
    copies = [None, None]
    copies[0] = pltpu.async_copy(feats[0].at[pl.ds(base, _EPW)], bufs[0],
                                 sems[0])

    tabs_v = [tb0, tb1, tb2, tb3]
    tcopies = [pltpu.async_copy(t, tv, semt)
               for t, tv in zip([t0, t1, t2, t3], tabs_v)]
    for c in tcopies:
        c.wait()
    lane_i = lax.iota(jnp.int32, _L)
    lanes = lane_i.astype(jnp.float32)
    s_list = []
    for f in range(_NFEAT):
        tv = tabs_v[f]
        s = jnp.zeros((_L,), jnp.float32)
        for i in range(_ROWS):
            r = tv[i, pl.ds(0, _L)] + tv[i, pl.ds(_L, _L)]
            for sh in (8, 4, 2, 1):
                r = r + r.at[lane_i ^ sh].get(mode="promise_in_bounds")
            s = jnp.where(lane_i == i, r, s)
        s_list.append(s * lanes)

    accs = tuple(jnp.zeros((_L,), jnp.float32) for _ in range(_UN))
    for f in range(_NFEAT):
        b = f % 2
        if f + 1 < _NFEAT:
            nb = (f + 1) % 2
            copies[nb] = pltpu.async_copy(
                feats[f + 1].at[pl.ds(base, _EPW)], bufs[nb], sems[nb])
        copies[b].wait()
        s = s_list[f]
        buf = bufs[b]

        @plsc.parallel_loop(0, _EPW, step=_UN * _L, carry=accs)
        def body(i, accs, buf=buf, s=s):
            new = []
            for j in range(_UN):
                v = buf[pl.ds(i + j * _L, _L)]
                idx = lax.bitcast_convert_type(v + jnp.float32(8388608.0),
                                               jnp.int32)
                new.append(accs[j] + s.at[idx].get(mode="promise_in_bounds"))
            return tuple(new)

        accs = body

    acc = accs[0]
    for j in range(1, _UN):
        acc = acc + accs[j]

    acc_v[:] = acc
    pltpu.sync_copy(acc_v, out.at[wid])


def kernel(bond_type, stereo, is_conjugated, is_in_ring,
           table_bond_type, table_stereo, table_is_conjugated,
           table_is_in_ring):
    mesh = plsc.VectorSubcoreMesh(core_axis_name="c", subcore_axis_name="s")
    run = pl.kernel(
        _sc_body,
        out_type=jax.ShapeDtypeStruct((_NW, _L), jnp.float32),
        mesh=mesh,
        scratch_types=[
            pltpu.VMEM((_EPW,), jnp.float32),
            pltpu.VMEM((_EPW,), jnp.float32),
            pltpu.VMEM((_ROWS, _DIM), jnp.float32),
            pltpu.VMEM((_ROWS, _DIM), jnp.float32),
            pltpu.VMEM((_ROWS, _DIM), jnp.float32),
            pltpu.VMEM((_ROWS, _DIM), jnp.float32),
            pltpu.VMEM((_L,), jnp.float32),
            pltpu.SemaphoreType.DMA,
            pltpu.SemaphoreType.DMA,
            pltpu.SemaphoreType.DMA,
        ],
    )
    partials = run(bond_type, stereo, is_conjugated, is_in_ring,
                   table_bond_type, table_stereo, table_is_conjugated,
                   table_is_in_ring)
    return jnp.sum(partials)
